# baseline (speedup 1.0000x reference)
import jax
import jax.numpy as jnp
from jax import lax
from jax.experimental import pallas as pl
from jax.experimental.pallas import tpu as pltpu
from jax.experimental.pallas import tpu_sc as plsc

N = 100000
D = 128
B = 16384

NC = 2
NS = 16
NW = NC * NS
PW = B // NW
CP = 64
NCHUNK = PW // CP
ROWS = 2 * CP
DCH = D // 16
NBUF = 2


def _rsqrt_f32(x):
    i = plsc.bitcast(x, jnp.int32)
    i = jnp.int32(0x5F3759DF) - (i >> 1)
    y = plsc.bitcast(i, jnp.float32)
    xh = x * jnp.float32(0.5)
    for _ in range(3):
        y = y * (jnp.float32(1.5) - xh * y * y)
    return y


def _make_body():
    def body(w_hbm, idxr_hbm, scale_hbm, out_hbm, *refs):
        idx_v = refs[0]
        bufs = refs[1:1 + NBUF]
        scale_v = refs[1 + NBUF]
        out_v = refs[2 + NBUF]
        acc_v = refs[3 + NBUF]
        sems = refs[4 + NBUF:4 + 2 * NBUF]

        wid = lax.axis_index("s") * NC + lax.axis_index("c")

        pltpu.sync_copy(idxr_hbm.at[pl.ds(wid * NCHUNK, NCHUNK)], idx_v)
        pltpu.sync_copy(scale_hbm, scale_v)
        scale = jnp.exp(scale_v[...])

        def start(c):
            return pltpu.async_copy(
                w_hbm.at[idx_v.at[c]], bufs[c % NBUF], sems[c % NBUF])

        def compute(c, rows):
            @plsc.parallel_loop(0, CP, 1, unroll=2)
            def pair(p):
                r0 = 2 * p
                a = rows[r0, pl.ds(0, 16)] - rows[r0 + 1, pl.ds(0, 16)]
                acc0 = a * a
                a = rows[r0, pl.ds(16, 16)] - rows[r0 + 1, pl.ds(16, 16)]
                acc1 = a * a
                for ch in range(2, DCH, 2):
                    a = (rows[r0, pl.ds(16 * ch, 16)]
                         - rows[r0 + 1, pl.ds(16 * ch, 16)])
                    acc0 = acc0 + a * a
                    a = (rows[r0, pl.ds(16 * ch + 16, 16)]
                         - rows[r0 + 1, pl.ds(16 * ch + 16, 16)])
                    acc1 = acc1 + a * a
                acc_v[pl.ds(p * 16, 16)] = acc0 + acc1

            lane16 = lax.broadcasted_iota(jnp.int32, (16,), 0) * 16
            @plsc.parallel_loop(0, CP // 16, 1)
            def group(g):
                base = lane16 + g * 256
                tot = plsc.load_gather(acc_v, [base])
                for t in range(1, 16):
                    tot = tot + plsc.load_gather(acc_v, [base + t])
                x = tot + jnp.float32(1e-12)
                d = x * _rsqrt_f32(x) * scale
                out_v[pl.ds(c * CP + g * 16, 16)] = d

        handles = {c: start(c) for c in range(min(NBUF - 1, NCHUNK))}
        for c in range(NCHUNK):
            if c + NBUF - 1 < NCHUNK:
                handles[c + NBUF - 1] = start(c + NBUF - 1)
            handles.pop(c).wait()
            compute(c, bufs[c % NBUF])

        pltpu.sync_copy(out_v, out_hbm.at[pl.ds(wid * PW, PW)])

    return body


@jax.jit
def _run(w, idxr, scale16):
    mesh = plsc.VectorSubcoreMesh(
        core_axis_name="c", subcore_axis_name="s",
        num_cores=NC, num_subcores=NS)
    scratch = [pltpu.VMEM((NCHUNK, ROWS), jnp.int32)]
    scratch += [pltpu.VMEM((ROWS, D), jnp.float32) for _ in range(NBUF)]
    scratch += [pltpu.VMEM((16,), jnp.float32),
                pltpu.VMEM((PW,), jnp.float32),
                pltpu.VMEM((CP * 16,), jnp.float32)]
    scratch += [pltpu.SemaphoreType.DMA for _ in range(NBUF)]
    return pl.kernel(
        _make_body(),
        out_type=jax.ShapeDtypeStruct((B,), jnp.float32),
        mesh=mesh,
        compiler_params=pltpu.CompilerParams(needs_layout_passes=False),
        scratch_types=scratch,
    )(w, idxr, scale16)


def kernel(idx, w, scale_log):
    idxr = idx.astype(jnp.int32).reshape(NW * NCHUNK, ROWS)
    scale16 = jnp.broadcast_to(scale_log.astype(jnp.float32), (16,))
    return _run(w, idxr, scale16)


# speedup vs baseline: 1.0211x; 1.0211x over previous
import jax
import jax.numpy as jnp
from jax import lax
from jax.experimental import pallas as pl
from jax.experimental.pallas import tpu as pltpu
from jax.experimental.pallas import tpu_sc as plsc

N = 100000
D = 128
B = 16384

NC = 2
NS = 16
NW = NC * NS
PW = B // NW
CP = 64
NCHUNK = PW // CP
ROWS = 2 * CP
DCH = D // 16
NBUF = 3


def _rsqrt_f32(x):
    i = plsc.bitcast(x, jnp.int32)
    i = jnp.int32(0x5F3759DF) - (i >> 1)
    y = plsc.bitcast(i, jnp.float32)
    xh = x * jnp.float32(0.5)
    for _ in range(3):
        y = y * (jnp.float32(1.5) - xh * y * y)
    return y


def _make_body():
    def body(w_hbm, idxr_hbm, scale_hbm, out_hbm, *refs):
        idx_v = refs[0]
        bufs = refs[1:1 + NBUF]
        scale_v = refs[1 + NBUF]
        out_v = refs[2 + NBUF]
        acc_v = refs[3 + NBUF]
        sems = refs[4 + NBUF:4 + 2 * NBUF]

        wid = lax.axis_index("s") * NC + lax.axis_index("c")

        pltpu.sync_copy(idxr_hbm.at[pl.ds(wid * NCHUNK, NCHUNK)], idx_v)
        pltpu.sync_copy(scale_hbm, scale_v)
        scale = jnp.exp(scale_v[...])

        def start(c):
            return pltpu.async_copy(
                w_hbm.at[idx_v.at[c]], bufs[c % NBUF], sems[c % NBUF])

        def compute(c, rows):
            @plsc.parallel_loop(0, CP, 1, unroll=2)
            def pair(p):
                r0 = 2 * p
                a = rows[r0, pl.ds(0, 16)] - rows[r0 + 1, pl.ds(0, 16)]
                acc0 = a * a
                a = rows[r0, pl.ds(16, 16)] - rows[r0 + 1, pl.ds(16, 16)]
                acc1 = a * a
                for ch in range(2, DCH, 2):
                    a = (rows[r0, pl.ds(16 * ch, 16)]
                         - rows[r0 + 1, pl.ds(16 * ch, 16)])
                    acc0 = acc0 + a * a
                    a = (rows[r0, pl.ds(16 * ch + 16, 16)]
                         - rows[r0 + 1, pl.ds(16 * ch + 16, 16)])
                    acc1 = acc1 + a * a
                acc_v[pl.ds(p * 16, 16)] = acc0 + acc1

            lane16 = lax.broadcasted_iota(jnp.int32, (16,), 0) * 16
            @plsc.parallel_loop(0, CP // 16, 1)
            def group(g):
                base = lane16 + g * 256
                tot = plsc.load_gather(acc_v, [base])
                for t in range(1, 16):
                    tot = tot + plsc.load_gather(acc_v, [base + t])
                x = tot + jnp.float32(1e-12)
                d = x * _rsqrt_f32(x) * scale
                out_v[pl.ds(c * CP + g * 16, 16)] = d

        handles = {c: start(c) for c in range(min(NBUF - 1, NCHUNK))}
        for c in range(NCHUNK):
            if c + NBUF - 1 < NCHUNK:
                handles[c + NBUF - 1] = start(c + NBUF - 1)
            handles.pop(c).wait()
            compute(c, bufs[c % NBUF])

        pltpu.sync_copy(out_v, out_hbm.at[pl.ds(wid * PW, PW)])

    return body


@jax.jit
def _run(w, idxr, scale16):
    mesh = plsc.VectorSubcoreMesh(
        core_axis_name="c", subcore_axis_name="s",
        num_cores=NC, num_subcores=NS)
    scratch = [pltpu.VMEM((NCHUNK, ROWS), jnp.int32)]
    scratch += [pltpu.VMEM((ROWS, D), jnp.float32) for _ in range(NBUF)]
    scratch += [pltpu.VMEM((16,), jnp.float32),
                pltpu.VMEM((PW,), jnp.float32),
                pltpu.VMEM((CP * 16,), jnp.float32)]
    scratch += [pltpu.SemaphoreType.DMA for _ in range(NBUF)]
    return pl.kernel(
        _make_body(),
        out_type=jax.ShapeDtypeStruct((B,), jnp.float32),
        mesh=mesh,
        compiler_params=pltpu.CompilerParams(needs_layout_passes=False),
        scratch_types=scratch,
    )(w, idxr, scale16)


def kernel(idx, w, scale_log):
    idxr = idx.astype(jnp.int32).reshape(NW * NCHUNK, ROWS)
    scale16 = jnp.broadcast_to(scale_log.astype(jnp.float32), (16,))
    return _run(w, idxr, scale16)


# speedup vs baseline: 1.0220x; 1.0009x over previous
import jax
import jax.numpy as jnp
from jax import lax
from jax.experimental import pallas as pl
from jax.experimental.pallas import tpu as pltpu
from jax.experimental.pallas import tpu_sc as plsc

N = 100000
D = 128
B = 16384

NC = 2
NS = 16
NW = NC * NS
PW = B // NW
CP = 64
NCHUNK = PW // CP
ROWS = 2 * CP
DCH = D // 16
NBUF = 3


def _rsqrt_f32(x):
    i = plsc.bitcast(x, jnp.int32)
    i = jnp.int32(0x5F3759DF) - (i >> 1)
    y = plsc.bitcast(i, jnp.float32)
    xh = x * jnp.float32(0.5)
    for _ in range(3):
        y = y * (jnp.float32(1.5) - xh * y * y)
    return y


def _make_body():
    def body(w_hbm, idxr_hbm, scale_hbm, out_hbm, *refs):
        idx_v = refs[0]
        bufs = refs[1:1 + NBUF]
        scale_v = refs[1 + NBUF]
        out_v = refs[2 + NBUF]
        acc_v = refs[3 + NBUF]
        sems = refs[4 + NBUF:4 + 2 * NBUF]

        wid = lax.axis_index("s") * NC + lax.axis_index("c")

        pltpu.sync_copy(idxr_hbm.at[pl.ds(wid * NCHUNK, NCHUNK)], idx_v)
        pltpu.sync_copy(scale_hbm, scale_v)
        scale = jnp.exp(scale_v[...])

        def start(c):
            return pltpu.async_copy(
                w_hbm.at[idx_v.at[c]], bufs[c % NBUF], sems[c % NBUF])

        def compute(c, rows):
            @plsc.parallel_loop(0, CP, 1, unroll=2, carry=jnp.int32(0))
            def pair(p, j):
                r0 = 2 * p
                a = rows[r0, pl.ds(0, 16)] - rows[r0 + 1, pl.ds(0, 16)]
                acc0 = a * a
                a = rows[r0, pl.ds(16, 16)] - rows[r0 + 1, pl.ds(16, 16)]
                acc1 = a * a
                for ch in range(2, DCH, 2):
                    a = (rows[r0, pl.ds(16 * ch, 16)]
                         - rows[r0 + 1, pl.ds(16 * ch, 16)])
                    acc0 = acc0 + a * a
                    a = (rows[r0, pl.ds(16 * ch + 16, 16)]
                         - rows[r0 + 1, pl.ds(16 * ch + 16, 16)])
                    acc1 = acc1 + a * a
                acc_v[pl.ds(p * 16, 16)] = acc0 + acc1
                return j + 1

            lane16 = lax.broadcasted_iota(jnp.int32, (16,), 0) * 16
            @plsc.parallel_loop(0, CP // 16, 1)
            def group(g):
                base = lane16 + (g * 256 + pair - CP)
                tot = plsc.load_gather(acc_v, [base])
                for t in range(1, 16):
                    tot = tot + plsc.load_gather(acc_v, [base + t])
                x = tot + jnp.float32(1e-12)
                d = x * _rsqrt_f32(x) * scale
                out_v[pl.ds(c * CP + g * 16, 16)] = d

        handles = {c: start(c) for c in range(min(NBUF - 1, NCHUNK))}
        for c in range(NCHUNK):
            if c + NBUF - 1 < NCHUNK:
                handles[c + NBUF - 1] = start(c + NBUF - 1)
            handles.pop(c).wait()
            compute(c, bufs[c % NBUF])

        pltpu.sync_copy(out_v, out_hbm.at[pl.ds(wid * PW, PW)])

    return body


@jax.jit
def _run(w, idxr, scale16):
    mesh = plsc.VectorSubcoreMesh(
        core_axis_name="c", subcore_axis_name="s",
        num_cores=NC, num_subcores=NS)
    scratch = [pltpu.VMEM((NCHUNK, ROWS), jnp.int32)]
    scratch += [pltpu.VMEM((ROWS, D), jnp.float32) for _ in range(NBUF)]
    scratch += [pltpu.VMEM((16,), jnp.float32),
                pltpu.VMEM((PW,), jnp.float32),
                pltpu.VMEM((CP * 16,), jnp.float32)]
    scratch += [pltpu.SemaphoreType.DMA for _ in range(NBUF)]
    return pl.kernel(
        _make_body(),
        out_type=jax.ShapeDtypeStruct((B,), jnp.float32),
        mesh=mesh,
        compiler_params=pltpu.CompilerParams(needs_layout_passes=False),
        scratch_types=scratch,
    )(w, idxr, scale16)


def kernel(idx, w, scale_log):
    idxr = idx.astype(jnp.int32).reshape(NW * NCHUNK, ROWS)
    scale16 = jnp.broadcast_to(scale_log.astype(jnp.float32), (16,))
    return _run(w, idxr, scale16)
